# Initial kernel scaffold; baseline (speedup 1.0000x reference)
import jax
import jax.numpy as jnp
from jax.experimental import pallas as pl


def kernel(pcld_input, kpts_pre_input, cpt_pre_input, seg_pre_input):
    raise NotImplementedError("write your pallas kernel here")


# speedup vs baseline: 8.6149x; 8.6149x over previous
import jax
import jax.numpy as jnp
from jax.experimental import pallas as pl

_NK = 8
_K = _NK + 1
_NCAND = 10
_SIGMA = 1.0


def _pose_kernel(offs_ref, pcld_ref, seg_ref, out_ref):
    offs = offs_ref[0]
    p = pcld_ref[0]
    seg = seg_ref[0]
    n = offs.shape[1]

    mask = seg[1:2] > seg[0:1]

    m_rows = []
    cx_rows = []
    cy_rows = []
    cz_rows = []
    for k in range(_K):
        x = offs[3 * k:3 * k + 1]
        y = offs[3 * k + 1:3 * k + 2]
        z = offs[3 * k + 2:3 * k + 3]
        nrm = jnp.sqrt(x * x + y * y + z * z)
        m_rows.append(jnp.where(mask, nrm, jnp.float32(1e9)))
        cx_rows.append(x + p[0:1])
        cy_rows.append(y + p[1:2])
        cz_rows.append(z + p[2:3])
    m = jnp.concatenate(m_rows, axis=0)
    cx = jnp.concatenate(cx_rows, axis=0)
    cy = jnp.concatenate(cy_rows, axis=0)
    cz = jnp.concatenate(cz_rows, axis=0)

    iota = jax.lax.broadcasted_iota(jnp.int32, (_K, n), 1)
    sel_x = []
    sel_y = []
    sel_z = []
    for _ in range(_NCAND):
        rowmin = jnp.min(m, axis=1, keepdims=True)
        idx = jnp.min(jnp.where(m == rowmin, iota, n), axis=1,
                      keepdims=True)
        sel = iota == idx
        sel_x.append(jnp.sum(jnp.where(sel, cx, 0.0), axis=1, keepdims=True))
        sel_y.append(jnp.sum(jnp.where(sel, cy, 0.0), axis=1, keepdims=True))
        sel_z.append(jnp.sum(jnp.where(sel, cz, 0.0), axis=1, keepdims=True))
        m = jnp.where(sel, jnp.float32(jnp.inf), m)

    gx = jnp.concatenate(sel_x, axis=1)
    gy = jnp.concatenate(sel_y, axis=1)
    gz = jnp.concatenate(sel_z, axis=1)

    mx = jnp.mean(gx, axis=1, keepdims=True)
    my = jnp.mean(gy, axis=1, keepdims=True)
    mz = jnp.mean(gz, axis=1, keepdims=True)
    sx = jnp.sqrt(jnp.mean((gx - mx) ** 2, axis=1, keepdims=True))
    sy = jnp.sqrt(jnp.mean((gy - my) ** 2, axis=1, keepdims=True))
    sz = jnp.sqrt(jnp.mean((gz - mz) ** 2, axis=1, keepdims=True))
    eps = jnp.float32(1e-9)
    inlier = ((jnp.abs(gx - mx) <= _SIGMA * sx + eps)
              & (jnp.abs(gy - my) <= _SIGMA * sy + eps)
              & (jnp.abs(gz - mz) <= _SIGMA * sz + eps))
    w = inlier.astype(jnp.float32)
    wsum = jnp.sum(w, axis=1, keepdims=True) + jnp.float32(1e-8)
    ox = jnp.sum(gx * w, axis=1, keepdims=True) / wsum
    oy = jnp.sum(gy * w, axis=1, keepdims=True) / wsum
    oz = jnp.sum(gz * w, axis=1, keepdims=True) / wsum
    out_ref[0] = jnp.concatenate([ox, oy, oz], axis=1)


def kernel(pcld_input, kpts_pre_input, cpt_pre_input, seg_pre_input):
    b, n, nk, _ = kpts_pre_input.shape
    assert nk == _NK
    offs = jnp.concatenate([kpts_pre_input, cpt_pre_input], axis=2)
    offs_t = offs.transpose(0, 2, 3, 1).reshape(b, 3 * _K, n)
    pcld_t = pcld_input.transpose(0, 2, 1)
    seg_t = seg_pre_input.transpose(0, 2, 1)

    out = pl.pallas_call(
        _pose_kernel,
        grid=(b,),
        in_specs=[
            pl.BlockSpec((1, 3 * _K, n), lambda i: (i, 0, 0)),
            pl.BlockSpec((1, 3, n), lambda i: (i, 0, 0)),
            pl.BlockSpec((1, 2, n), lambda i: (i, 0, 0)),
        ],
        out_specs=pl.BlockSpec((1, _K, 3), lambda i: (i, 0, 0)),
        out_shape=jax.ShapeDtypeStruct((b, _K, 3), jnp.float32),
    )(offs_t, pcld_t, seg_t)
    return out
